# Initial kernel scaffold; baseline (speedup 1.0000x reference)
import jax
import jax.numpy as jnp
from jax.experimental import pallas as pl


def kernel(input, receptive_field, adj, W, Wk, Wq, bn_gamma, bn_beta):
    raise NotImplementedError("write your pallas kernel here")


# speedup vs baseline: 36.2501x; 36.2501x over previous
import functools

import jax
import jax.numpy as jnp
import numpy as np
from jax import lax
from jax.experimental import pallas as pl
from jax.experimental.pallas import tpu as pltpu
from jax.experimental.pallas import tpu_sc as plsc

N = 10000
F = 128
DEG = 32
NB = DEG * DEG
EPS = 1e-5
NEG = np.float32(-1e30)
NEG2 = np.float32(-3e38)
NC, NS = 2, 16
NW = NC * NS
GROUPS = N // 16
_FULL = GROUPS % NW

P = N * DEG
CH = 128
SUP = 8
CROWS = 2560
RPW = CROWS // NW

_MESH = plsc.VectorSubcoreMesh(
    core_axis_name="c", subcore_axis_name="s", num_cores=NC, num_subcores=NS
)


def _worker_id():
    return lax.axis_index("s") * NC + lax.axis_index("c")


def _num_groups(wid):
    hi = GROUPS // NW + 1
    return jnp.where(wid < _FULL, hi, hi - 1).astype(jnp.int32)


def _mm_body(x_ref, w_ref, kq_ref, nh_ref, kqo_ref):
    x = x_ref[...]
    nh_ref[...] = jnp.dot(x, w_ref[...], preferred_element_type=jnp.float32)
    kqo_ref[...] = jnp.dot(x, kq_ref[...], preferred_element_type=jnp.float32)


def _matmul(x, w0, wkq):
    blk = 2000
    return pl.pallas_call(
        _mm_body,
        grid=(N // blk,),
        in_specs=[
            pl.BlockSpec((blk, F), lambda i: (i, 0)),
            pl.BlockSpec((F, F), lambda i: (0, 0)),
            pl.BlockSpec((F, 2), lambda i: (0, 0)),
        ],
        out_specs=[
            pl.BlockSpec((blk, F), lambda i: (i, 0)),
            pl.BlockSpec((blk, 2), lambda i: (i, 0)),
        ],
        out_shape=[
            jax.ShapeDtypeStruct((N, F), jnp.float32),
            jax.ShapeDtypeStruct((N, 2), jnp.float32),
        ],
    )(x, w0, wkq)


def _flat_gather_body(tab_hbm, idx_hbm, out_hbm, idx_v, buf_v, sem):
    wid = _worker_id()
    r_base = wid * RPW

    def super_body(t, carry):
        r0 = r_base + t * SUP
        pltpu.sync_copy(idx_hbm.at[pl.ds(r0, SUP)], idx_v)
        copies = [
            pltpu.async_copy(tab_hbm.at[idx_v.at[b]], buf_v.at[b], sem)
            for b in range(SUP)
        ]
        for c in copies:
            c.wait()
        pltpu.sync_copy(buf_v, out_hbm.at[pl.ds(r0, SUP)])
        return carry

    lax.fori_loop(0, RPW // SUP, super_body, 0)


def _make_flat_gather(dtype):
    return functools.partial(
        pl.kernel,
        out_type=jax.ShapeDtypeStruct((CROWS, CH), dtype),
        mesh=_MESH,
        scratch_types=[
            pltpu.VMEM((SUP, CH), jnp.int32),
            pltpu.VMEM((SUP, CH), dtype),
            pltpu.SemaphoreType.DMA,
        ],
    )(_flat_gather_body)


_flat_gather_f32 = _make_flat_gather(jnp.float32)
_flat_gather_i32 = _make_flat_gather(jnp.int32)


def _smw_body(key_ref, qg_ref, rf_ref, w_ref):
    att = key_ref[...] * qg_ref[...]
    att = jnp.where(rf_ref[...] != N - 1, att, NEG)
    m = jnp.max(att, axis=1, keepdims=True)
    e = jnp.exp(att - m)
    w_ref[...] = e / jnp.sum(e, axis=1, keepdims=True)


def _softmax_w(key2d, qg, rf):
    blk = 2000
    return pl.pallas_call(
        _smw_body,
        grid=(N // blk,),
        in_specs=[
            pl.BlockSpec((blk, 1), lambda i: (i, 0)),
            pl.BlockSpec((blk, DEG), lambda i: (i, 0)),
            pl.BlockSpec((blk, DEG), lambda i: (i, 0)),
        ],
        out_specs=pl.BlockSpec((blk, DEG), lambda i: (i, 0)),
        out_shape=jax.ShapeDtypeStruct((N, DEG), jnp.float32),
    )(key2d, qg, rf)


def _wsum_body(rf_hbm, w_hbm, nh_hbm, pre_hbm,
               rf_v, w_v, own_v, gbuf, out_v, sem):
    wid = _worker_id()

    def group_body(t, carry):
        g = wid + NW * t
        r0 = g * 16
        pltpu.sync_copy(rf_hbm.at[pl.ds(r0, 16)], rf_v)
        pltpu.sync_copy(w_hbm.at[pl.ds(r0, 16)], w_v)
        pltpu.sync_copy(nh_hbm.at[pl.ds(r0, 16)], own_v)
        copies = [
            pltpu.async_copy(nh_hbm.at[rf_v.at[r]], gbuf.at[r], sem)
            for r in range(16)
        ]
        for c in copies:
            c.wait()

        def row_body(r, carry2):
            acc = [own_v[r, pl.ds(c * 16, 16)] for c in range(8)]
            wrow = [w_v[r, pl.ds(h * 16, 16)] for h in range(2)]
            for j in range(DEG):
                wj = wrow[j // 16][j % 16]
                for c in range(8):
                    acc[c] = acc[c] + wj * gbuf[r, j, pl.ds(c * 16, 16)]
            for c in range(8):
                out_v[r, pl.ds(c * 16, 16)] = acc[c]
            return carry2

        lax.fori_loop(0, 16, row_body, 0)
        pltpu.sync_copy(out_v, pre_hbm.at[pl.ds(r0, 16)])
        return carry

    lax.fori_loop(0, _num_groups(wid), group_body, 0)


_wsum = functools.partial(
    pl.kernel,
    out_type=jax.ShapeDtypeStruct((N, F), jnp.float32),
    mesh=_MESH,
    scratch_types=[
        pltpu.VMEM((16, DEG), jnp.int32),
        pltpu.VMEM((16, DEG), jnp.float32),
        pltpu.VMEM((16, F), jnp.float32),
        pltpu.VMEM((16, DEG, F), jnp.float32),
        pltpu.VMEM((16, F), jnp.float32),
        pltpu.SemaphoreType.DMA,
    ],
)(_wsum_body)


def _colsum_body(pre_ref, out_ref):
    @pl.when(pl.program_id(0) == 0)
    def _():
        out_ref[...] = jnp.zeros_like(out_ref)

    x = pre_ref[...]
    out_ref[0, :] += jnp.sum(x, axis=0)
    out_ref[1, :] += jnp.sum(x * x, axis=0)


def _colsum(pre):
    blk = 2000
    return pl.pallas_call(
        _colsum_body,
        grid=(N // blk,),
        in_specs=[pl.BlockSpec((blk, F), lambda i: (i, 0))],
        out_specs=pl.BlockSpec((2, F), lambda i: (0, 0)),
        out_shape=jax.ShapeDtypeStruct((2, F), jnp.float32),
    )(pre)


def _bn_body(pre_ref, stats_ref, gamma_ref, beta_ref, out_ref):
    s = stats_ref[...]
    mean = s[0, :] / N
    var = s[1, :] / N - mean * mean
    scale = gamma_ref[...] * lax.rsqrt(var + EPS)
    y = (pre_ref[...] - mean[None, :]) * scale[None, :] + beta_ref[...][None, :]
    out_ref[...] = jnp.maximum(y, 0.0)


def _bn(pre, stats, gamma, beta):
    blk = 2000
    return pl.pallas_call(
        _bn_body,
        grid=(N // blk,),
        in_specs=[
            pl.BlockSpec((blk, F), lambda i: (i, 0)),
            pl.BlockSpec((2, F), lambda i: (0, 0)),
            pl.BlockSpec((F,), lambda i: (0,)),
            pl.BlockSpec((F,), lambda i: (0,)),
        ],
        out_specs=pl.BlockSpec((blk, F), lambda i: (i, 0)),
        out_shape=jax.ShapeDtypeStruct((N, F), jnp.float32),
    )(pre, stats, gamma, beta)


def _comb_body(adj_ref, qa_ref, out_ref):
    qa_i = lax.bitcast_convert_type(qa_ref[...], jnp.int32)
    z = jnp.zeros((adj_ref.shape[0], F - 2 * DEG), jnp.int32)
    out_ref[...] = jnp.concatenate([adj_ref[...], qa_i, z], axis=1)


def _combine(adj, qa):
    blk = 2000
    return pl.pallas_call(
        _comb_body,
        grid=(N // blk,),
        in_specs=[
            pl.BlockSpec((blk, DEG), lambda i: (i, 0)),
            pl.BlockSpec((blk, DEG), lambda i: (i, 0)),
        ],
        out_specs=pl.BlockSpec((blk, F), lambda i: (i, 0)),
        out_shape=jax.ShapeDtypeStruct((N, F), jnp.int32),
    )(adj, qa)


def _nbr_body(rf_hbm, comb_hbm, nbq_hbm, rf_v, nb_buf, sem):
    wid = _worker_id()

    def group_body(t, carry):
        g = wid + NW * t
        r0 = g * 16
        pltpu.sync_copy(rf_hbm.at[pl.ds(r0, 16)], rf_v)
        copies = [
            pltpu.async_copy(comb_hbm.at[rf_v.at[r]], nb_buf.at[r], sem)
            for r in range(16)
        ]
        for c in copies:
            c.wait()
        pltpu.sync_copy(nb_buf, nbq_hbm.at[pl.ds(r0, 16)])
        return carry

    lax.fori_loop(0, _num_groups(wid), group_body, 0)


_nbr = functools.partial(
    pl.kernel,
    out_type=jax.ShapeDtypeStruct((N, DEG, F), jnp.int32),
    mesh=_MESH,
    scratch_types=[
        pltpu.VMEM((16, DEG), jnp.int32),
        pltpu.VMEM((16, DEG, F), jnp.int32),
        pltpu.SemaphoreType.DMA,
    ],
)(_nbr_body)


def _topk_body(key_ref, nbq_ref, sel_ref, a_ref, *, blk):
    ids = nbq_ref[:, :, 0:DEG]
    q = lax.bitcast_convert_type(nbq_ref[:, :, DEG:2 * DEG], jnp.float32)
    a = key_ref[...][:, :, None] * q
    a_ref[...] = jnp.where(ids != N - 1, a, NEG)
    iota3 = (
        lax.broadcasted_iota(jnp.int32, (blk, DEG, DEG), 1) * DEG
        + lax.broadcasted_iota(jnp.int32, (blk, DEG, DEG), 2)
    )
    lane = lax.broadcasted_iota(jnp.int32, (blk, DEG), 1)
    sel_ref[...] = jnp.zeros((blk, DEG), jnp.int32)

    def it_body(i, carry):
        av = a_ref[...]
        m = jnp.max(av, axis=(1, 2), keepdims=True)
        pos = jnp.min(jnp.where(av >= m, iota3, NB), axis=(1, 2), keepdims=True)
        a_ref[...] = jnp.where(iota3 == pos, NEG2, av)
        pos2 = pos.reshape(pos.shape[0], 1)
        sel_ref[...] += jnp.where(lane == i, pos2, 0)
        return carry

    lax.fori_loop(0, DEG, it_body, 0)

    sel = sel_ref[...]
    row = pl.program_id(0) * blk + lax.broadcasted_iota(jnp.int32, sel.shape, 0)
    j = lax.shift_right_logical(sel, 5)
    d = lax.bitwise_and(sel, DEG - 1)
    sel_ref[...] = row * (DEG * F) + j * F + d


def _topk(key2d, nbq):
    blk = 400
    return pl.pallas_call(
        functools.partial(_topk_body, blk=blk),
        grid=(N // blk,),
        in_specs=[
            pl.BlockSpec((blk, 1), lambda i: (i, 0)),
            pl.BlockSpec((blk, DEG, F), lambda i: (i, 0, 0)),
        ],
        out_specs=pl.BlockSpec((blk, DEG), lambda i: (i, 0)),
        out_shape=jax.ShapeDtypeStruct((N, DEG), jnp.int32),
        scratch_shapes=[pltpu.VMEM((blk, DEG, DEG), jnp.float32)],
    )(key2d, nbq)


def _pad_idx(idx_flat):
    return jnp.concatenate(
        [idx_flat, jnp.zeros(CROWS * CH - P, jnp.int32)]
    ).reshape(CROWS, CH)


def kernel(input, receptive_field, adj, W, Wk, Wq, bn_gamma, bn_beta):
    x = input
    rf0 = receptive_field[0]
    wkq = jnp.concatenate([Wk, Wq], axis=1)
    nh, kq = _matmul(x, W[0], wkq)
    key2d = kq[:, 0:1]
    query = kq[:, 1] + jnp.float32(0)

    qg = _flat_gather_f32(query, _pad_idx(rf0.reshape(-1)))
    qg = qg.reshape(-1)[:P].reshape(N, DEG)
    w = _softmax_w(key2d, qg, rf0)
    pre = _wsum(rf0, w, nh)
    stats = _colsum(pre)
    final_h = _bn(pre, stats, bn_gamma, bn_beta)

    qa = _flat_gather_f32(query, _pad_idx(adj.reshape(-1)))
    qa = qa.reshape(-1)[:P].reshape(N, DEG)
    comb = _combine(adj, qa)
    nbq = _nbr(rf0, comb)
    selg = _topk(key2d, nbq)
    expand = _flat_gather_i32(
        nbq.reshape(N * DEG * F), _pad_idx(selg.reshape(-1))
    )
    expand = expand.reshape(-1)[:P].reshape(N, DEG)

    new_rf = jnp.concatenate(
        [receptive_field, expand[None].astype(receptive_field.dtype)], axis=0
    )
    return final_h, new_rf


# speedup vs baseline: 113.6258x; 3.1345x over previous
import functools

import jax
import jax.numpy as jnp
import numpy as np
from jax import lax
from jax.experimental import pallas as pl
from jax.experimental.pallas import tpu as pltpu
from jax.experimental.pallas import tpu_sc as plsc

N = 10000
F = 128
DEG = 32
NB = DEG * DEG
EPS = 1e-5
NEG = np.float32(-1e30)
NEG2 = np.float32(-3e38)
NC, NS = 2, 16
NW = NC * NS
GROUPS = N // 16
_FULL = GROUPS % NW

P = N * DEG
CH = 128
SUP = 8
CROWS = 2560
RPW = CROWS // NW

_MESH = plsc.VectorSubcoreMesh(
    core_axis_name="c", subcore_axis_name="s", num_cores=NC, num_subcores=NS
)


def _worker_id():
    return lax.axis_index("s") * NC + lax.axis_index("c")


def _num_groups(wid):
    hi = GROUPS // NW + 1
    return jnp.where(wid < _FULL, hi, hi - 1).astype(jnp.int32)


def _mm_body(x_ref, w_ref, kq_ref, nh_ref, kqo_ref):
    x = x_ref[...]
    nh_ref[...] = jnp.dot(x, w_ref[...], preferred_element_type=jnp.float32)
    kqo_ref[...] = jnp.dot(x, kq_ref[...], preferred_element_type=jnp.float32)


def _matmul(x, w0, wkq):
    blk = 2000
    return pl.pallas_call(
        _mm_body,
        grid=(N // blk,),
        in_specs=[
            pl.BlockSpec((blk, F), lambda i: (i, 0)),
            pl.BlockSpec((F, F), lambda i: (0, 0)),
            pl.BlockSpec((F, 2), lambda i: (0, 0)),
        ],
        out_specs=[
            pl.BlockSpec((blk, F), lambda i: (i, 0)),
            pl.BlockSpec((blk, 2), lambda i: (i, 0)),
        ],
        out_shape=[
            jax.ShapeDtypeStruct((N, F), jnp.float32),
            jax.ShapeDtypeStruct((N, 2), jnp.float32),
        ],
    )(x, w0, wkq)


def _flat_gather_body(tab_hbm, idx_hbm, out_hbm, idx_v, buf_v, sem):
    wid = _worker_id()
    r_base = wid * RPW

    def super_body(t, carry):
        r0 = r_base + t * SUP
        pltpu.sync_copy(idx_hbm.at[pl.ds(r0, SUP)], idx_v)
        copies = [
            pltpu.async_copy(tab_hbm.at[idx_v.at[b]], buf_v.at[b], sem)
            for b in range(SUP)
        ]
        for c in copies:
            c.wait()
        pltpu.sync_copy(buf_v, out_hbm.at[pl.ds(r0, SUP)])
        return carry

    lax.fori_loop(0, RPW // SUP, super_body, 0)


def _make_flat_gather(dtype):
    return functools.partial(
        pl.kernel,
        out_type=jax.ShapeDtypeStruct((CROWS, CH), dtype),
        mesh=_MESH,
        scratch_types=[
            pltpu.VMEM((SUP, CH), jnp.int32),
            pltpu.VMEM((SUP, CH), dtype),
            pltpu.SemaphoreType.DMA,
        ],
    )(_flat_gather_body)


_flat_gather_f32 = _make_flat_gather(jnp.float32)
_flat_gather_i32 = _make_flat_gather(jnp.int32)


def _smw_body(key_ref, qg_ref, rf_ref, w_ref):
    att = key_ref[...] * qg_ref[...]
    att = jnp.where(rf_ref[...] != N - 1, att, NEG)
    m = jnp.max(att, axis=1, keepdims=True)
    e = jnp.exp(att - m)
    w_ref[...] = e / jnp.sum(e, axis=1, keepdims=True)


def _softmax_w(key2d, qg, rf):
    blk = 2000
    return pl.pallas_call(
        _smw_body,
        grid=(N // blk,),
        in_specs=[
            pl.BlockSpec((blk, 1), lambda i: (i, 0)),
            pl.BlockSpec((blk, DEG), lambda i: (i, 0)),
            pl.BlockSpec((blk, DEG), lambda i: (i, 0)),
        ],
        out_specs=pl.BlockSpec((blk, DEG), lambda i: (i, 0)),
        out_shape=jax.ShapeDtypeStruct((N, DEG), jnp.float32),
    )(key2d, qg, rf)


def _wsum_body(rf_hbm, w_hbm, nh_hbm, pre_hbm,
               rf_v, w_v, own_v, gbuf, out_v, sem):
    wid = _worker_id()

    def group_body(t, carry):
        g = wid + NW * t
        r0 = g * 16
        pltpu.sync_copy(rf_hbm.at[pl.ds(r0, 16)], rf_v)
        pltpu.sync_copy(w_hbm.at[pl.ds(r0, 16)], w_v)
        pltpu.sync_copy(nh_hbm.at[pl.ds(r0, 16)], own_v)
        copies = [
            pltpu.async_copy(nh_hbm.at[rf_v.at[r]], gbuf.at[r], sem)
            for r in range(16)
        ]
        for c in copies:
            c.wait()

        def row_body(r, carry2):
            acc = [own_v[r, pl.ds(c * 16, 16)] for c in range(8)]
            wrow = [w_v[r, pl.ds(h * 16, 16)] for h in range(2)]
            for j in range(DEG):
                wj = wrow[j // 16][j % 16]
                for c in range(8):
                    acc[c] = acc[c] + wj * gbuf[r, j, pl.ds(c * 16, 16)]
            for c in range(8):
                out_v[r, pl.ds(c * 16, 16)] = acc[c]
            return carry2

        lax.fori_loop(0, 16, row_body, 0)
        pltpu.sync_copy(out_v, pre_hbm.at[pl.ds(r0, 16)])
        return carry

    lax.fori_loop(0, _num_groups(wid), group_body, 0)


_wsum = functools.partial(
    pl.kernel,
    out_type=jax.ShapeDtypeStruct((N, F), jnp.float32),
    mesh=_MESH,
    scratch_types=[
        pltpu.VMEM((16, DEG), jnp.int32),
        pltpu.VMEM((16, DEG), jnp.float32),
        pltpu.VMEM((16, F), jnp.float32),
        pltpu.VMEM((16, DEG, F), jnp.float32),
        pltpu.VMEM((16, F), jnp.float32),
        pltpu.SemaphoreType.DMA,
    ],
)(_wsum_body)


def _colsum_body(pre_ref, out_ref):
    @pl.when(pl.program_id(0) == 0)
    def _():
        out_ref[...] = jnp.zeros_like(out_ref)

    x = pre_ref[...]
    out_ref[0, :] += jnp.sum(x, axis=0)
    out_ref[1, :] += jnp.sum(x * x, axis=0)


def _colsum(pre):
    blk = 2000
    return pl.pallas_call(
        _colsum_body,
        grid=(N // blk,),
        in_specs=[pl.BlockSpec((blk, F), lambda i: (i, 0))],
        out_specs=pl.BlockSpec((2, F), lambda i: (0, 0)),
        out_shape=jax.ShapeDtypeStruct((2, F), jnp.float32),
    )(pre)


def _bn_body(pre_ref, stats_ref, gamma_ref, beta_ref, out_ref):
    s = stats_ref[...]
    mean = s[0, :] / N
    var = s[1, :] / N - mean * mean
    scale = gamma_ref[...] * lax.rsqrt(var + EPS)
    y = (pre_ref[...] - mean[None, :]) * scale[None, :] + beta_ref[...][None, :]
    out_ref[...] = jnp.maximum(y, 0.0)


def _bn(pre, stats, gamma, beta):
    blk = 2000
    return pl.pallas_call(
        _bn_body,
        grid=(N // blk,),
        in_specs=[
            pl.BlockSpec((blk, F), lambda i: (i, 0)),
            pl.BlockSpec((2, F), lambda i: (0, 0)),
            pl.BlockSpec((F,), lambda i: (0,)),
            pl.BlockSpec((F,), lambda i: (0,)),
        ],
        out_specs=pl.BlockSpec((blk, F), lambda i: (i, 0)),
        out_shape=jax.ShapeDtypeStruct((N, F), jnp.float32),
    )(pre, stats, gamma, beta)


CW = F


def _comb_body(adj_ref, qa_ref, out_ref):
    qa_i = lax.bitcast_convert_type(qa_ref[...], jnp.int32)
    z = jnp.zeros((adj_ref.shape[0], CW - 2 * DEG), jnp.int32)
    out_ref[...] = jnp.concatenate([adj_ref[...], qa_i, z], axis=1)


def _combine(adj, qa):
    blk = 2000
    return pl.pallas_call(
        _comb_body,
        grid=(N // blk,),
        in_specs=[
            pl.BlockSpec((blk, DEG), lambda i: (i, 0)),
            pl.BlockSpec((blk, DEG), lambda i: (i, 0)),
        ],
        out_specs=pl.BlockSpec((blk, CW), lambda i: (i, 0)),
        out_shape=jax.ShapeDtypeStruct((N, CW), jnp.int32),
    )(adj, qa)


def _nbr_body(rf_hbm, comb_hbm, nbq_hbm, rf_v, nb_buf, sem):
    wid = _worker_id()

    def group_body(t, carry):
        g = wid + NW * t
        r0 = g * 16
        pltpu.sync_copy(rf_hbm.at[pl.ds(r0, 16)], rf_v)
        copies = [
            pltpu.async_copy(comb_hbm.at[rf_v.at[r]], nb_buf.at[r], sem)
            for r in range(16)
        ]
        for c in copies:
            c.wait()
        pltpu.sync_copy(nb_buf, nbq_hbm.at[pl.ds(r0, 16)])
        return carry

    lax.fori_loop(0, _num_groups(wid), group_body, 0)


_nbr = functools.partial(
    pl.kernel,
    out_type=jax.ShapeDtypeStruct((N, DEG, CW), jnp.int32),
    mesh=_MESH,
    scratch_types=[
        pltpu.VMEM((16, DEG), jnp.int32),
        pltpu.VMEM((16, DEG, CW), jnp.int32),
        pltpu.SemaphoreType.DMA,
    ],
)(_nbr_body)


def _topk_body(key_ref, nbq_ref, sel_ref, a_ref, *, blk):
    ids = nbq_ref[:, :, 0:DEG].reshape(blk, NB)
    q = lax.bitcast_convert_type(
        nbq_ref[:, :, DEG:2 * DEG], jnp.float32
    ).reshape(blk, NB)
    a = key_ref[...] * q
    a_ref[...] = jnp.where(ids != N - 1, a, NEG)
    iota2 = lax.broadcasted_iota(jnp.int32, (blk, NB), 1)
    lane = lax.broadcasted_iota(jnp.int32, (blk, DEG), 1)
    sel_ref[...] = jnp.zeros((blk, DEG), jnp.int32)

    def it_body(i, carry):
        av = a_ref[...]
        m = jnp.max(av, axis=1, keepdims=True)
        pos = jnp.min(jnp.where(av >= m, iota2, NB), axis=1, keepdims=True)
        a_ref[...] = jnp.where(iota2 == pos, NEG2, av)
        sel_ref[...] += jnp.where(lane == i, pos, 0)
        return carry

    lax.fori_loop(0, DEG, it_body, 0)

    sel = sel_ref[...]
    row = pl.program_id(0) * blk + lax.broadcasted_iota(jnp.int32, sel.shape, 0)
    j = lax.shift_right_logical(sel, 5)
    d = lax.bitwise_and(sel, DEG - 1)
    sel_ref[...] = row * (DEG * CW) + j * CW + d


def _topk(key2d, nbq):
    blk = 400
    return pl.pallas_call(
        functools.partial(_topk_body, blk=blk),
        grid=(N // blk,),
        in_specs=[
            pl.BlockSpec((blk, 1), lambda i: (i, 0)),
            pl.BlockSpec((blk, DEG, CW), lambda i: (i, 0, 0)),
        ],
        out_specs=pl.BlockSpec((blk, DEG), lambda i: (i, 0)),
        out_shape=jax.ShapeDtypeStruct((N, DEG), jnp.int32),
        scratch_shapes=[pltpu.VMEM((blk, NB), jnp.float32)],
    )(key2d, nbq)


def _pad_idx(idx_flat):
    return jnp.concatenate(
        [idx_flat, jnp.zeros(CROWS * CH - P, jnp.int32)]
    ).reshape(CROWS, CH)


def kernel(input, receptive_field, adj, W, Wk, Wq, bn_gamma, bn_beta):
    x = input
    rf0 = receptive_field[0]
    wkq = jnp.concatenate([Wk, Wq], axis=1)
    nh, kq = _matmul(x, W[0], wkq)
    key2d = kq[:, 0:1]
    query = kq[:, 1] + jnp.float32(0)

    qg = _flat_gather_f32(query, _pad_idx(rf0.reshape(-1)))
    qg = qg.reshape(-1)[:P].reshape(N, DEG)
    w = _softmax_w(key2d, qg, rf0)
    pre = _wsum(rf0, w, nh)
    stats = _colsum(pre)
    final_h = _bn(pre, stats, bn_gamma, bn_beta)

    qa = _flat_gather_f32(query, _pad_idx(adj.reshape(-1)))
    qa = qa.reshape(-1)[:P].reshape(N, DEG)
    comb = _combine(adj, qa)
    nbq = _nbr(rf0, comb)
    selg = _topk(key2d, nbq)
    expand = _flat_gather_i32(
        nbq.reshape(N * DEG * CW), _pad_idx(selg.reshape(-1))
    )
    expand = expand.reshape(-1)[:P].reshape(N, DEG)

    new_rf = jnp.concatenate(
        [receptive_field, expand[None].astype(receptive_field.dtype)], axis=0
    )
    return final_h, new_rf
